# baseline (speedup 1.0000x reference)
import jax
import jax.numpy as jnp
from jax.experimental import pallas as pl


def _pick_bm(n: int, target: int) -> int:
    bm = 8
    for cand in range(8, target + 1, 8):
        if n % cand == 0:
            bm = cand
    return bm


def _first_layer_kernel(a_ref, xb_ref, x_ref, ab_ref, hb_ref, s_ref):
    a = a_ref[...]
    ab = a.astype(jnp.bfloat16)
    ab_ref[...] = ab
    h = jnp.dot(ab, xb_ref[...], preferred_element_type=jnp.float32)
    hb_ref[...] = h.astype(jnp.bfloat16)
    s_ref[...] = x_ref[...] + h


def _mid_layer_kernel(ab_ref, hb_ref, s_ref, ho_ref, so_ref):
    h = jnp.dot(ab_ref[...], hb_ref[...], preferred_element_type=jnp.float32)
    ho_ref[...] = h.astype(jnp.bfloat16)
    so_ref[...] = s_ref[...] + h


def _last_layer_kernel(ab_ref, hb_ref, s_ref, out_ref):
    h = jnp.dot(ab_ref[...], hb_ref[...], preferred_element_type=jnp.float32)
    out_ref[...] = (s_ref[...] + h) * 0.25


def kernel(node_embeddings, adj):
    n, d = node_embeddings.shape
    x = node_embeddings
    xb = x.astype(jnp.bfloat16)

    bm1 = _pick_bm(n, 400)
    bm2 = _pick_bm(n, 800)

    row_block = lambda bm: pl.BlockSpec((bm, n), lambda i: (i, 0))
    full_rhs = pl.BlockSpec((n, d), lambda i: (0, 0))
    out_block = lambda bm: pl.BlockSpec((bm, d), lambda i: (i, 0))

    ab, h1b, s1 = pl.pallas_call(
        _first_layer_kernel,
        grid=(n // bm1,),
        in_specs=[row_block(bm1), full_rhs, out_block(bm1)],
        out_specs=[row_block(bm1), out_block(bm1), out_block(bm1)],
        out_shape=[
            jax.ShapeDtypeStruct((n, n), jnp.bfloat16),
            jax.ShapeDtypeStruct((n, d), jnp.bfloat16),
            jax.ShapeDtypeStruct((n, d), jnp.float32),
        ],
    )(adj, xb, x)

    h2b, s2 = pl.pallas_call(
        _mid_layer_kernel,
        grid=(n // bm2,),
        in_specs=[row_block(bm2), full_rhs, out_block(bm2)],
        out_specs=[out_block(bm2), out_block(bm2)],
        out_shape=[
            jax.ShapeDtypeStruct((n, d), jnp.bfloat16),
            jax.ShapeDtypeStruct((n, d), jnp.float32),
        ],
    )(ab, h1b, s1)

    out = pl.pallas_call(
        _last_layer_kernel,
        grid=(n // bm2,),
        in_specs=[row_block(bm2), full_rhs, out_block(bm2)],
        out_specs=out_block(bm2),
        out_shape=jax.ShapeDtypeStruct((n, d), jnp.float32),
    )(ab, h2b, s2)

    return out


# speedup vs baseline: 1.0139x; 1.0139x over previous
import jax
import jax.numpy as jnp
from jax.experimental import pallas as pl


def _pick_bm(n: int, target: int) -> int:
    bm = 8
    for cand in range(8, target + 1, 8):
        if n % cand == 0:
            bm = cand
    return bm


def _first_layer_kernel(a_ref, xb_ref, x_ref, ab_ref, hb_ref, s_ref):
    a = a_ref[...]
    ab = a.astype(jnp.bfloat16)
    ab_ref[...] = ab
    h = jnp.dot(ab, xb_ref[...], preferred_element_type=jnp.float32)
    hb_ref[...] = h.astype(jnp.bfloat16)
    s_ref[...] = x_ref[...] + h


def _mid_layer_kernel(ab_ref, hb_ref, s_ref, ho_ref, so_ref):
    h = jnp.dot(ab_ref[...], hb_ref[...], preferred_element_type=jnp.float32)
    ho_ref[...] = h.astype(jnp.bfloat16)
    so_ref[...] = s_ref[...] + h


def _last_layer_kernel(ab_ref, hb_ref, s_ref, out_ref):
    h = jnp.dot(ab_ref[...], hb_ref[...], preferred_element_type=jnp.float32)
    out_ref[...] = (s_ref[...] + h) * 0.25


def kernel(node_embeddings, adj):
    n, d = node_embeddings.shape
    x = node_embeddings
    xb = x.astype(jnp.bfloat16)

    bm1 = _pick_bm(n, 400)
    bm2 = _pick_bm(n, 1000)

    row_block = lambda bm: pl.BlockSpec((bm, n), lambda i: (i, 0))
    full_rhs = pl.BlockSpec((n, d), lambda i: (0, 0))
    out_block = lambda bm: pl.BlockSpec((bm, d), lambda i: (i, 0))

    ab, h1b, s1 = pl.pallas_call(
        _first_layer_kernel,
        grid=(n // bm1,),
        in_specs=[row_block(bm1), full_rhs, out_block(bm1)],
        out_specs=[row_block(bm1), out_block(bm1), out_block(bm1)],
        out_shape=[
            jax.ShapeDtypeStruct((n, n), jnp.bfloat16),
            jax.ShapeDtypeStruct((n, d), jnp.bfloat16),
            jax.ShapeDtypeStruct((n, d), jnp.float32),
        ],
    )(adj, xb, x)

    h2b, s2 = pl.pallas_call(
        _mid_layer_kernel,
        grid=(n // bm2,),
        in_specs=[row_block(bm2), full_rhs, out_block(bm2)],
        out_specs=[out_block(bm2), out_block(bm2)],
        out_shape=[
            jax.ShapeDtypeStruct((n, d), jnp.bfloat16),
            jax.ShapeDtypeStruct((n, d), jnp.float32),
        ],
    )(ab, h1b, s1)

    out = pl.pallas_call(
        _last_layer_kernel,
        grid=(n // bm2,),
        in_specs=[row_block(bm2), full_rhs, out_block(bm2)],
        out_specs=out_block(bm2),
        out_shape=jax.ShapeDtypeStruct((n, d), jnp.float32),
    )(ab, h2b, s2)

    return out
